# baseline (speedup 1.0000x reference)
import functools
import math

import jax
import jax.numpy as jnp
from jax import lax
from jax.experimental import pallas as pl
from jax.experimental.pallas import tpu as pltpu
from jax.experimental.pallas import tpu_sc as plsc

D = 64
K = 20
MARGIN = 0.5
NC = 2
NS = 16
NW = NC * NS
CHUNK = 32
L = 16


@functools.lru_cache(maxsize=None)
def _make_sc_kernel(batch: int, n_items: int):
    bpw = batch // NW
    nchunks = bpw // CHUNK
    half = n_items // 2
    mesh = plsc.VectorSubcoreMesh(core_axis_name="c", subcore_axis_name="s")
    logvals = [math.log(c * n_items / K + 1.0) for c in range(K + 1)]

    @functools.partial(
        pl.kernel,
        mesh=mesh,
        compiler_params=pltpu.CompilerParams(
            needs_layout_passes=False, use_tc_tiling_on_sc=True),
        out_type=jax.ShapeDtypeStruct((NW, 16), jnp.float32),
        scratch_types=[
            pltpu.VMEM((bpw,), jnp.int32),
            pltpu.VMEM((bpw,), jnp.int32),
            pltpu.VMEM((K, bpw), jnp.int32),
            pltpu.VMEM((bpw,), jnp.int32),
            pltpu.VMEM((bpw,), jnp.int32),
            pltpu.VMEM((K, bpw), jnp.int32),
            pltpu.VMEM((CHUNK, 2 * D), jnp.float32),
            pltpu.VMEM((CHUNK, 2 * D), jnp.float32),
            pltpu.VMEM((K, CHUNK, 2 * D), jnp.float32),
            pltpu.VMEM((32,), jnp.float32),
            pltpu.VMEM((16,), jnp.float32),
            pltpu.SemaphoreType.DMA,
        ],
    )
    def sc(uid_hbm, pid_hbm, nid_hbm, uemb_hbm, iemb_hbm, out_hbm,
           uid_v, pid_v, nid_v, urow_v, prow_v, nrow_v,
           u_v, p_v, n_v, logtab, o_v, sem):
        wid = lax.axis_index("s") * NC + lax.axis_index("c")
        base = wid * bpw
        pltpu.sync_copy(uid_hbm.at[pl.ds(base, bpw)], uid_v)
        pltpu.sync_copy(pid_hbm.at[pl.ds(base, bpw)], pid_v)
        pltpu.sync_copy(nid_hbm.at[:, pl.ds(base, bpw)], nid_v)

        lanes = lax.iota(jnp.int32, L)
        t0 = jnp.zeros((L,), jnp.float32)
        t1 = jnp.zeros((L,), jnp.float32)
        for c in range(L):
            t0 = jnp.where(lanes == c, jnp.float32(logvals[c]), t0)
        for c in range(L, K + 1):
            t1 = jnp.where(lanes == (c - L), jnp.float32(logvals[c]), t1)
        logtab[pl.ds(0, L)] = t0
        logtab[pl.ds(L, L)] = t1

        def rows_body(i, _):
            s = pl.ds(i * L, L)
            urow_v[s] = jax.lax.shift_right_logical(uid_v[s], 1)
            prow_v[s] = jax.lax.shift_right_logical(pid_v[s], 1)
            for k in range(K):
                nrow_v[k, s] = jax.lax.shift_right_logical(nid_v[k, s], 1)
            return 0
        lax.fori_loop(0, bpw // L, rows_body, 0)

        def chunk_body(ci, loss):
            off = ci * CHUNK
            cp_u = pltpu.async_copy(uemb_hbm.at[urow_v.at[pl.ds(off, CHUNK)]],
                                    u_v, sem)
            cp_p = pltpu.async_copy(iemb_hbm.at[prow_v.at[pl.ds(off, CHUNK)]],
                                    p_v, sem)
            cps = []
            for k in range(K):
                cps.append(pltpu.async_copy(
                    iemb_hbm.at[nrow_v.at[k, pl.ds(off, CHUNK)]],
                    n_v.at[k], sem))
            cp_u.wait()
            cp_p.wait()
            for cp in cps:
                cp.wait()

            for g in range(CHUNK // L):
                rowv = jnp.full((L,), g * L, jnp.int32) + lanes
                s = pl.ds(off + g * L, L)
                ucol = (uid_v[s] & 1) * D
                pcol = (pid_v[s] & 1) * D
                ncols = [(nid_v[k, s] & 1) * D for k in range(K)]
                kz = [jnp.full((L,), k, jnp.int32) for k in range(K)]

                def d_body(d, accs):
                    dp, dn = accs
                    dv = jnp.full((L,), 0, jnp.int32) + d
                    uval = plsc.load_gather(u_v, [rowv, ucol + dv])
                    pd = plsc.load_gather(p_v, [rowv, pcol + dv]) - uval
                    dp = dp + pd * pd
                    new_dn = []
                    for k in range(K):
                        nd = plsc.load_gather(
                            n_v, [kz[k], rowv, ncols[k] + dv]) - uval
                        new_dn.append(dn[k] + nd * nd)
                    return (dp, new_dn)

                zero = jnp.zeros((L,), jnp.float32)
                pos_dist, neg_dists = lax.fori_loop(
                    0, D, d_body, (zero, [zero] * K))

                thr = pos_dist + MARGIN
                closest = functools.reduce(jnp.minimum, neg_dists)
                cnt = jnp.zeros((L,), jnp.int32)
                for nd in neg_dists:
                    cnt = cnt + (thr > nd).astype(jnp.int32)
                lp = jnp.maximum(thr - closest, 0.0)
                logw = plsc.load_gather(logtab, [cnt])
                loss = loss + lp * logw
            return loss

        loss_v = lax.fori_loop(0, nchunks, chunk_body,
                               jnp.zeros((L,), jnp.float32))
        o_v[...] = jnp.broadcast_to(jnp.sum(loss_v), (16,))
        pltpu.sync_copy(o_v, out_hbm.at[wid])

    return sc


def kernel(user_ids, pos_item_ids, neg_item_ids, user_emb, item_emb):
    batch = user_ids.shape[0]
    n_items = item_emb.shape[0]
    sc = _make_sc_kernel(batch, n_items)
    partial = sc(user_ids, pos_item_ids, neg_item_ids.T,
                 user_emb.reshape(n_items // 2, 2 * D),
                 item_emb.reshape(n_items // 2, 2 * D))
    return partial[:, 0].sum()


# speedup vs baseline: 1.6753x; 1.6753x over previous
import functools
import math

import jax
import jax.numpy as jnp
from jax import lax
from jax.experimental import pallas as pl
from jax.experimental.pallas import tpu as pltpu
from jax.experimental.pallas import tpu_sc as plsc

D = 64
K = 20
MARGIN = 0.5
NC = 2
NS = 16
NW = NC * NS
CHUNK = 32
TCHUNK = 64
L = 16


@functools.lru_cache(maxsize=None)
def _make_user_rows_kernel(batch: int, n_users: int):
    bpw = batch // NW
    nchunks = bpw // TCHUNK
    mesh = plsc.VectorSubcoreMesh(core_axis_name="c", subcore_axis_name="s")

    @functools.partial(
        pl.kernel,
        mesh=mesh,
        compiler_params=pltpu.CompilerParams(
            needs_layout_passes=False, use_tc_tiling_on_sc=True),
        out_type=jax.ShapeDtypeStruct((batch // 2, 2 * D), jnp.float32),
        scratch_types=[
            pltpu.VMEM((bpw,), jnp.int32),
            pltpu.VMEM((bpw,), jnp.int32),
            pltpu.VMEM((TCHUNK, 8, D), jnp.float32),
            pltpu.VMEM((TCHUNK // 2, 2 * D), jnp.float32),
            pltpu.SemaphoreType.DMA,
        ],
    )
    def ka(uid_hbm, utab_hbm, out_hbm, uid_v, trow_v, tiles_v, pack_v, sem):
        wid = lax.axis_index("s") * NC + lax.axis_index("c")
        base = wid * bpw
        pltpu.sync_copy(uid_hbm.at[pl.ds(base, bpw)], uid_v)

        def rows_body(i, _):
            s = pl.ds(i * L, L)
            trow_v[s] = jax.lax.shift_right_logical(uid_v[s], 3)
            return 0
        lax.fori_loop(0, bpw // L, rows_body, 0)

        lanes = lax.iota(jnp.int32, L)

        def chunk_body(ci, _):
            off = ci * TCHUNK
            cps = []
            for g4 in range(TCHUNK // L):
                tv = trow_v[pl.ds(off + g4 * L, L)]
                for j in range(L):
                    t = jnp.max(jnp.where(lanes == j, tv, jnp.int32(0)))
                    t8 = pl.multiple_of(t * 8, 8)
                    cps.append(pltpu.async_copy(
                        utab_hbm.at[pl.ds(t8, 8), :],
                        tiles_v.at[g4 * L + j], sem))
            for cp in cps:
                cp.wait()
            for g in range(TCHUNK // L):
                pvec = lanes + g * L
                subrow = uid_v[pl.ds(off + g * L, L)] & 7
                dst_r = jax.lax.shift_right_logical(pvec, 1)
                dst_c0 = (pvec & 1) * D

                def d_body(d, _):
                    dv = jnp.full((L,), 0, jnp.int32) + d
                    val = plsc.load_gather(tiles_v, [pvec, subrow, dv])
                    plsc.store_scatter(pack_v, [dst_r, dst_c0 + dv], val)
                    return 0
                lax.fori_loop(0, D, d_body, 0)
            r0 = pl.multiple_of((base + off) // 2, 8)
            pltpu.sync_copy(
                pack_v, out_hbm.at[pl.ds(r0, TCHUNK // 2), :])
            return 0

        lax.fori_loop(0, nchunks, chunk_body, 0)

    return ka


@functools.lru_cache(maxsize=None)
def _make_loss_kernel(batch: int, n_items: int):
    bpw = batch // NW
    nchunks = bpw // CHUNK
    mesh = plsc.VectorSubcoreMesh(core_axis_name="c", subcore_axis_name="s")
    logvals = [math.log(c * n_items / K + 1.0) for c in range(K + 1)]

    @functools.partial(
        pl.kernel,
        mesh=mesh,
        compiler_params=pltpu.CompilerParams(
            needs_layout_passes=False, use_tc_tiling_on_sc=False),
        out_type=jax.ShapeDtypeStruct((NW, 16), jnp.float32),
        scratch_types=[
            pltpu.VMEM((bpw // 2, 2 * D), jnp.float32),
            pltpu.VMEM((bpw,), jnp.int32),
            pltpu.VMEM((K, bpw), jnp.int32),
            pltpu.VMEM((CHUNK, D), jnp.float32),
            pltpu.VMEM((K, CHUNK, D), jnp.float32),
            pltpu.VMEM((16,), jnp.float32),
            pltpu.SMEM((32,), jnp.float32),
            pltpu.SemaphoreType.DMA,
        ],
    )
    def kb(upack_hbm, pid_hbm, nid_hbm, iemb_hbm, out_hbm,
           u_v, pid_v, nid_v, p_v, n_v, o_v, logtab, sem):
        wid = lax.axis_index("s") * NC + lax.axis_index("c")
        base = wid * bpw
        pltpu.sync_copy(
            upack_hbm.at[pl.ds(pl.multiple_of(base // 2, 8), bpw // 2), :],
            u_v)
        pltpu.sync_copy(pid_hbm.at[pl.ds(base, bpw)], pid_v)
        pltpu.sync_copy(nid_hbm.at[:, pl.ds(base, bpw)], nid_v)
        for c in range(K + 1):
            logtab[c] = jnp.float32(logvals[c])

        def chunk_body(ci, loss):
            off = ci * CHUNK
            cp_p = pltpu.async_copy(iemb_hbm.at[pid_v.at[pl.ds(off, CHUNK)]],
                                    p_v, sem)
            cps = []
            for k in range(K):
                cps.append(pltpu.async_copy(
                    iemb_hbm.at[nid_v.at[k, pl.ds(off, CHUNK)]],
                    n_v.at[k], sem))
            cp_p.wait()
            for cp in cps:
                cp.wait()

            def pair_body(b, l):
                gb = off + b
                ur = jax.lax.shift_right_logical(gb, 1)
                uc = (gb & 1) * D
                uv = [u_v[ur, pl.ds(uc + 16 * c, 16)] for c in range(4)]
                pv = [p_v[b, pl.ds(16 * c, 16)] for c in range(4)]
                dp0 = uv[0] - pv[0]
                dp1 = uv[1] - pv[1]
                dp2 = uv[2] - pv[2]
                dp3 = uv[3] - pv[3]
                pos_dist = jnp.sum(dp0 * dp0 + dp1 * dp1
                                   + dp2 * dp2 + dp3 * dp3)
                thr = pos_dist + MARGIN
                nds = []
                for k in range(K):
                    d0 = uv[0] - n_v[k, b, pl.ds(0, 16)]
                    d1 = uv[1] - n_v[k, b, pl.ds(16, 16)]
                    d2 = uv[2] - n_v[k, b, pl.ds(32, 16)]
                    d3 = uv[3] - n_v[k, b, pl.ds(48, 16)]
                    nds.append(jnp.sum(d0 * d0 + d1 * d1 + d2 * d2 + d3 * d3))
                closest = functools.reduce(jnp.minimum, nds)
                cnt = jnp.int32(0)
                for nd in nds:
                    cnt = cnt + (thr > nd).astype(jnp.int32)
                lp = jnp.maximum(thr - closest, jnp.float32(0.0))
                return l + lp * logtab[cnt]

            return lax.fori_loop(0, CHUNK, pair_body, loss)

        loss = lax.fori_loop(0, nchunks, chunk_body, jnp.float32(0.0))
        o_v[...] = jnp.broadcast_to(loss, (16,))
        pltpu.sync_copy(o_v, out_hbm.at[wid])

    return kb


def kernel(user_ids, pos_item_ids, neg_item_ids, user_emb, item_emb):
    batch = user_ids.shape[0]
    n_users = user_emb.shape[0]
    n_items = item_emb.shape[0]
    ka = _make_user_rows_kernel(batch, n_users)
    upack = ka(user_ids, user_emb)
    kb = _make_loss_kernel(batch, n_items)
    partial = kb(upack, pos_item_ids, neg_item_ids.T, item_emb)
    return partial[:, 0].sum()
